# baseline (speedup 1.0000x reference)
import functools

import jax
import jax.numpy as jnp
from jax import lax
from jax.experimental import pallas as pl
from jax.experimental.pallas import tpu as pltpu
from jax.experimental.pallas import tpu_sc as plsc

B = 4
MODEL_DIM = 1024
HIGH_DIM = 4096
KEEP = 256
L = 2048

O_TILE = 512
L_TILE = 512

GROUP = 16
NGROUPS = (HIGH_DIM // GROUP) * L

NC = 2
NS = 16
NT = NC * NS
TPB = NT // B
CHUNK = NGROUPS // TPB
NV = CHUNK // 16
NBKT = 256
HSIZE = NBKT * 16
CAP = 512
NSEL = 512


def _mm1_body(x_ref, w1_ref, b1_ref, h_ref, gmax_ref):
    w = w1_ref[...]
    xt = x_ref[0]
    acc = jnp.dot(w, xt, preferred_element_type=jnp.float32)
    ht = acc + b1_ref[...]
    h_ref[0] = ht
    gmax_ref[...] = jnp.max(ht.reshape(O_TILE // GROUP, GROUP, L_TILE), axis=1)


def _matmul1(x, W1, b1c):
    grid = (B, HIGH_DIM // O_TILE, L // L_TILE)
    return pl.pallas_call(
        _mm1_body,
        grid=grid,
        in_specs=[
            pl.BlockSpec((1, MODEL_DIM, L_TILE), lambda b, o, l: (b, 0, l)),
            pl.BlockSpec((O_TILE, MODEL_DIM), lambda b, o, l: (o, 0)),
            pl.BlockSpec((O_TILE, 1), lambda b, o, l: (o, 0)),
        ],
        out_specs=[
            pl.BlockSpec((1, O_TILE, L_TILE), lambda b, o, l: (b, o, l)),
            pl.BlockSpec(
                (O_TILE // GROUP, L_TILE),
                lambda b, o, l: (b * (HIGH_DIM // O_TILE) + o, l)),
        ],
        out_shape=[
            jax.ShapeDtypeStruct((B, HIGH_DIM, L), jnp.float32),
            jax.ShapeDtypeStruct((B * (HIGH_DIM // GROUP), L), jnp.float32),
        ],
    )(x, W1, b1c)


def _select_body(gmax_hbm, cand_hbm, buf, hist, red, tmp, outb, shared):
    c = lax.axis_index("c")
    s = lax.axis_index("s")
    wid = c * NS + s
    b = wid // TPB
    t = wid % TPB
    base = t * CHUNK
    sbase = (s // (NS // 2)) * (NS // 2)

    lanes = lax.iota(jnp.int32, 16)
    ones = jnp.ones((16,), jnp.int32)
    zero16 = jnp.zeros((16,), jnp.int32)
    neg16 = jnp.full((16,), -1, jnp.int32)

    row0 = b * (HIGH_DIM // GROUP) + t * (CHUNK // L)
    pltpu.sync_copy(gmax_hbm.at[pl.ds(row0, CHUNK // L), :], buf)

    def keys_at(v):
        x = buf[v >> 7, pl.ds((v & 127) * 16, 16)]
        i = lax.bitcast_convert_type(x, jnp.int32)
        return jnp.where(i < 0, i ^ jnp.int32(0x7FFFFFFF), i)

    def zero_ref(ref):
        def body(i, carry):
            ref[pl.ds(i * 16, 16)] = zero16
            return carry
        lax.fori_loop(0, HSIZE // 16, body, 0)

    def reduce_team():
        pltpu.sync_copy(shared.at[pl.ds(sbase * HSIZE, HSIZE)], red)

        def row(j, carry):
            pltpu.sync_copy(shared.at[pl.ds((sbase + j) * HSIZE, HSIZE)], tmp)

            def addv(i, c2):
                red[pl.ds(i * 16, 16)] = (red[pl.ds(i * 16, 16)]
                                          + tmp[pl.ds(i * 16, 16)])
                return c2
            lax.fori_loop(0, HSIZE // 16, addv, 0)
            return carry
        lax.fori_loop(1, TPB, row, 0)

    def suffix_scan(init_cnt):
        def body(i, carry):
            suffix, bstar, chi = carry
            bkt = 255 - i
            cnt = jnp.sum(red[pl.ds(bkt * 16, 16)])
            nsuf = suffix + cnt
            crossed = jnp.logical_and(nsuf >= KEEP, bstar < 0)
            bstar = jnp.where(crossed, bkt, bstar)
            chi = jnp.where(crossed, suffix, chi)
            return (nsuf, bstar, chi)
        return lax.fori_loop(
            0, NBKT, body, (init_cnt, jnp.int32(-1), jnp.int32(0)))

    zero_ref(hist)

    def h1(v4, carry):
        for u in range(4):
            k = keys_at(v4 * 4 + u)
            bkt = lax.shift_right_arithmetic(k, 24) + 128
            plsc.addupdate_scatter(hist, [bkt * 16 + lanes], ones)
        return carry
    lax.fori_loop(0, NV // 4, h1, 0)

    pltpu.sync_copy(hist, shared.at[pl.ds(s * HSIZE, HSIZE)])
    plsc.subcore_barrier()
    reduce_team()
    plsc.subcore_barrier()
    _, b1, chi1 = suffix_scan(jnp.int32(0))

    zero_ref(hist)

    def h2(v4, carry):
        for u in range(4):
            k = keys_at(v4 * 4 + u)
            in_b1 = (lax.shift_right_arithmetic(k, 24) + 128) == b1
            b2 = lax.shift_right_logical(k, 16) & 0xFF
            plsc.addupdate_scatter(hist, [b2 * 16 + lanes], ones, mask=in_b1)
        return carry
    lax.fori_loop(0, NV // 4, h2, 0)

    pltpu.sync_copy(hist, shared.at[pl.ds(s * HSIZE, HSIZE)])
    plsc.subcore_barrier()
    reduce_team()
    _, b2s, _ = suffix_scan(chi1)

    theta = lax.shift_left(b1 - 128, 24) + lax.shift_left(b2s, 16)

    def init_out(i, carry):
        outb[pl.ds(i * 16, 16)] = neg16
        return carry
    lax.fori_loop(0, CAP // 16, init_out, 0)

    def comp(v4, off):
        for u in range(4):
            v = v4 * 4 + u
            k = keys_at(v)
            m = k >= theta
            cnt = jnp.sum(m.astype(jnp.int32))
            can = off <= CAP - 16

            @pl.when(jnp.logical_and(can, cnt > 0))
            def _():
                plsc.store_compressed(outb.at[pl.ds(off, 16)],
                                      base + v * 16 + lanes, mask=m)

            off = off + jnp.where(can, cnt, jnp.int32(0))
        return off
    lax.fori_loop(0, NV // 4, comp, jnp.int32(0))

    pltpu.sync_copy(outb, cand_hbm.at[pl.ds(wid * CAP, CAP)])


@functools.partial(
    pl.kernel,
    mesh=plsc.VectorSubcoreMesh(core_axis_name="c", subcore_axis_name="s"),
    out_type=jax.ShapeDtypeStruct((NT * CAP,), jnp.int32),
    scratch_types=[
        pltpu.VMEM((CHUNK // L, L), jnp.float32),
        pltpu.VMEM((HSIZE,), jnp.int32),
        pltpu.VMEM((HSIZE,), jnp.int32),
        pltpu.VMEM((HSIZE,), jnp.int32),
        pltpu.VMEM((CAP,), jnp.int32),
        pltpu.VMEM_SHARED((NS * HSIZE,), jnp.int32),
    ],
    compiler_params=pltpu.CompilerParams(needs_layout_passes=False,
                                         use_tc_tiling_on_sc=True),
)
def _select_groups(gmax_hbm, cand_hbm, buf, hist, red, tmp, outb, shared):
    _select_body(gmax_hbm, cand_hbm, buf, hist, red, tmp, outb, shared)


def _mm2_body(w2_ref, o_ref, v_ref, l_ref, b2_ref, y_ref):
    o = o_ref[0]
    v = v_ref[0]
    li = l_ref[0]
    ch = lax.broadcasted_iota(jnp.int32, (KEEP, HIGH_DIM), 1)
    e = jnp.where(o == ch, 1.0, 0.0)
    a = lax.dot_general(e, w2_ref[...], (((1,), (1,)), ((), ())),
                        preferred_element_type=jnp.float32)
    cols = lax.broadcasted_iota(jnp.int32, (KEEP, L), 1)
    p = jnp.where(li == cols, v, 0.0)
    acc = lax.dot_general(a, p, (((0,), (0,)), ((), ())),
                          preferred_element_type=jnp.float32)
    y_ref[0] = acc + b2_ref[...]


def _matmul2(W2, o_idx, vals, l_idx, b2c):
    return pl.pallas_call(
        _mm2_body,
        grid=(B,),
        in_specs=[
            pl.BlockSpec((MODEL_DIM, HIGH_DIM), lambda b: (0, 0)),
            pl.BlockSpec((1, KEEP, 1), lambda b: (b, 0, 0)),
            pl.BlockSpec((1, KEEP, 1), lambda b: (b, 0, 0)),
            pl.BlockSpec((1, KEEP, 1), lambda b: (b, 0, 0)),
            pl.BlockSpec((MODEL_DIM, 1), lambda b: (0, 0)),
        ],
        out_specs=pl.BlockSpec((1, MODEL_DIM, L), lambda b: (b, 0, 0)),
        out_shape=jax.ShapeDtypeStruct((B, MODEL_DIM, L), jnp.float32),
    )(W2, o_idx[:, :, None], vals[:, :, None], l_idx[:, :, None], b2c)


def kernel(x, W1, b1, W2, b2):
    b1c = b1[:, None]
    b2c = b2[:, None]

    h, gmax = _matmul1(x, W1, b1c)

    cand = _select_groups(gmax)
    slots = cand.reshape(B, TPB * CAP)
    top_slots, _ = lax.top_k(slots, NSEL)
    valid = top_slots >= 0
    gid = jnp.maximum(top_slots, 0)
    gm = gid // L
    gl = gid % L

    hf = h.reshape(B, HIGH_DIM * L)
    cf = ((gm[:, :, None] * GROUP + jnp.arange(GROUP)[None, None, :]) * L
          + gl[:, :, None])
    cf = jnp.where(valid[:, :, None], cf, HIGH_DIM * L).reshape(B, NSEL * GROUP)
    cf = jnp.sort(cf, axis=1)
    cv = jnp.take_along_axis(hf, jnp.minimum(cf, HIGH_DIM * L - 1), axis=1)
    cv = jnp.where(cf >= HIGH_DIM * L, -jnp.inf, cv)
    vals, pos = lax.top_k(cv, KEEP)
    idx = jnp.take_along_axis(cf, pos, axis=1)
    o_idx = idx // L
    l_idx = idx % L

    return _matmul2(W2, o_idx, vals, l_idx, b2c)


# speedup vs baseline: 1.0076x; 1.0076x over previous
import functools

import jax
import jax.numpy as jnp
from jax import lax
from jax.experimental import pallas as pl
from jax.experimental.pallas import tpu as pltpu
from jax.experimental.pallas import tpu_sc as plsc

B = 4
MODEL_DIM = 1024
HIGH_DIM = 4096
KEEP = 256
L = 2048

O_TILE = 512
L_TILE = 512

GROUP = 16
NGROUPS = (HIGH_DIM // GROUP) * L

NC = 2
NS = 16
NT = NC * NS
TPB = NT // B
CHUNK = NGROUPS // TPB
NV = CHUNK // 16
NBKT = 256
HSIZE = NBKT * 16
CAP = 512
NSEL = 512


def _mm1_body(x_ref, w1_ref, b1_ref, h_ref, gmax_ref):
    w = w1_ref[...]
    xt = x_ref[0]
    acc = jnp.dot(w, xt, preferred_element_type=jnp.float32)
    ht = acc + b1_ref[...]
    h_ref[0] = ht
    gmax_ref[...] = jnp.max(ht.reshape(O_TILE // GROUP, GROUP, L_TILE), axis=1)


def _matmul1(x, W1, b1c):
    grid = (B, HIGH_DIM // O_TILE, L // L_TILE)
    return pl.pallas_call(
        _mm1_body,
        grid=grid,
        in_specs=[
            pl.BlockSpec((1, MODEL_DIM, L_TILE), lambda b, o, l: (b, 0, l)),
            pl.BlockSpec((O_TILE, MODEL_DIM), lambda b, o, l: (o, 0)),
            pl.BlockSpec((O_TILE, 1), lambda b, o, l: (o, 0)),
        ],
        out_specs=[
            pl.BlockSpec((1, O_TILE, L_TILE), lambda b, o, l: (b, o, l)),
            pl.BlockSpec(
                (O_TILE // GROUP, L_TILE),
                lambda b, o, l: (b * (HIGH_DIM // O_TILE) + o, l)),
        ],
        out_shape=[
            jax.ShapeDtypeStruct((B, HIGH_DIM, L), jnp.float32),
            jax.ShapeDtypeStruct((B * (HIGH_DIM // GROUP), L), jnp.float32),
        ],
    )(x, W1, b1c)


def _select_body(gmax_hbm, cand_hbm, buf, hist, red, tmp, outb, shared):
    c = lax.axis_index("c")
    s = lax.axis_index("s")
    wid = c * NS + s
    b = wid // TPB
    t = wid % TPB
    base = t * CHUNK
    sbase = (s // (NS // 2)) * (NS // 2)

    lanes = lax.iota(jnp.int32, 16)
    ones = jnp.ones((16,), jnp.int32)
    zero16 = jnp.zeros((16,), jnp.int32)
    neg16 = jnp.full((16,), -1, jnp.int32)

    row0 = b * (HIGH_DIM // GROUP) + t * (CHUNK // L)
    pltpu.sync_copy(gmax_hbm.at[pl.ds(row0, CHUNK // L), :], buf)

    def keys_at(v):
        x = buf[v >> 7, pl.ds((v & 127) * 16, 16)]
        i = lax.bitcast_convert_type(x, jnp.int32)
        return jnp.where(i < 0, i ^ jnp.int32(0x7FFFFFFF), i)

    def zero_ref(ref):
        def body(i, carry):
            ref[pl.ds(i * 16, 16)] = zero16
            return carry
        lax.fori_loop(0, HSIZE // 16, body, 0)

    def reduce_team():
        pltpu.sync_copy(shared.at[pl.ds(sbase * HSIZE, HSIZE)], red)

        def row(j, carry):
            pltpu.sync_copy(shared.at[pl.ds((sbase + j) * HSIZE, HSIZE)], tmp)

            def addv(i, c2):
                red[pl.ds(i * 16, 16)] = (red[pl.ds(i * 16, 16)]
                                          + tmp[pl.ds(i * 16, 16)])
                return c2
            lax.fori_loop(0, HSIZE // 16, addv, 0)
            return carry
        lax.fori_loop(1, TPB, row, 0)

    def suffix_scan(init_cnt):
        def body(i, carry):
            suffix, bstar, chi = carry
            bkt = 255 - i
            cnt = jnp.sum(red[pl.ds(bkt * 16, 16)])
            nsuf = suffix + cnt
            crossed = jnp.logical_and(nsuf >= KEEP, bstar < 0)
            bstar = jnp.where(crossed, bkt, bstar)
            chi = jnp.where(crossed, suffix, chi)
            return (nsuf, bstar, chi)
        return lax.fori_loop(
            0, NBKT, body, (init_cnt, jnp.int32(-1), jnp.int32(0)))

    zero_ref(hist)

    def h1(v, carry):
        k = keys_at(v)
        bkt = lax.shift_right_arithmetic(k, 24) + 128
        plsc.addupdate_scatter(hist, [bkt * 16 + lanes], ones)
        return carry
    lax.fori_loop(0, NV, h1, 0)

    pltpu.sync_copy(hist, shared.at[pl.ds(s * HSIZE, HSIZE)])
    plsc.subcore_barrier()
    reduce_team()
    plsc.subcore_barrier()
    _, b1, chi1 = suffix_scan(jnp.int32(0))

    zero_ref(hist)

    def h2(v, carry):
        k = keys_at(v)
        in_b1 = (lax.shift_right_arithmetic(k, 24) + 128) == b1
        b2 = lax.shift_right_logical(k, 16) & 0xFF
        plsc.addupdate_scatter(hist, [b2 * 16 + lanes], ones, mask=in_b1)
        return carry
    lax.fori_loop(0, NV, h2, 0)

    pltpu.sync_copy(hist, shared.at[pl.ds(s * HSIZE, HSIZE)])
    plsc.subcore_barrier()
    reduce_team()
    _, b2s, _ = suffix_scan(chi1)

    theta = lax.shift_left(b1 - 128, 24) + lax.shift_left(b2s, 16)

    def init_out(i, carry):
        outb[pl.ds(i * 16, 16)] = neg16
        return carry
    lax.fori_loop(0, CAP // 16, init_out, 0)

    def comp(v, off):
        k = keys_at(v)
        m = k >= theta
        cnt = jnp.sum(m.astype(jnp.int32))
        can = off <= CAP - 16

        @pl.when(jnp.logical_and(can, cnt > 0))
        def _():
            plsc.store_compressed(outb.at[pl.ds(off, 16)],
                                  base + v * 16 + lanes, mask=m)

        return off + jnp.where(can, cnt, jnp.int32(0))
    lax.fori_loop(0, NV, comp, jnp.int32(0))

    pltpu.sync_copy(outb, cand_hbm.at[pl.ds(wid * CAP, CAP)])


@functools.partial(
    pl.kernel,
    mesh=plsc.VectorSubcoreMesh(core_axis_name="c", subcore_axis_name="s"),
    out_type=jax.ShapeDtypeStruct((NT * CAP,), jnp.int32),
    scratch_types=[
        pltpu.VMEM((CHUNK // L, L), jnp.float32),
        pltpu.VMEM((HSIZE,), jnp.int32),
        pltpu.VMEM((HSIZE,), jnp.int32),
        pltpu.VMEM((HSIZE,), jnp.int32),
        pltpu.VMEM((CAP,), jnp.int32),
        pltpu.VMEM_SHARED((NS * HSIZE,), jnp.int32),
    ],
    compiler_params=pltpu.CompilerParams(needs_layout_passes=False,
                                         use_tc_tiling_on_sc=True),
)
def _select_groups(gmax_hbm, cand_hbm, buf, hist, red, tmp, outb, shared):
    _select_body(gmax_hbm, cand_hbm, buf, hist, red, tmp, outb, shared)


def _mm2_body(w2_ref, o_ref, v_ref, l_ref, b2_ref, y_ref):
    o = o_ref[0]
    v = v_ref[0]
    li = l_ref[0]
    ch = lax.broadcasted_iota(jnp.int32, (KEEP, HIGH_DIM), 1)
    e = jnp.where(o == ch, 1.0, 0.0)
    a = lax.dot_general(e, w2_ref[...], (((1,), (1,)), ((), ())),
                        preferred_element_type=jnp.float32)
    cols = lax.broadcasted_iota(jnp.int32, (KEEP, L), 1)
    p = jnp.where(li == cols, v, 0.0)
    acc = lax.dot_general(a, p, (((0,), (0,)), ((), ())),
                          preferred_element_type=jnp.float32)
    y_ref[0] = acc + b2_ref[...]


def _matmul2(W2, o_idx, vals, l_idx, b2c):
    return pl.pallas_call(
        _mm2_body,
        grid=(B,),
        in_specs=[
            pl.BlockSpec((MODEL_DIM, HIGH_DIM), lambda b: (0, 0)),
            pl.BlockSpec((1, KEEP, 1), lambda b: (b, 0, 0)),
            pl.BlockSpec((1, KEEP, 1), lambda b: (b, 0, 0)),
            pl.BlockSpec((1, KEEP, 1), lambda b: (b, 0, 0)),
            pl.BlockSpec((MODEL_DIM, 1), lambda b: (0, 0)),
        ],
        out_specs=pl.BlockSpec((1, MODEL_DIM, L), lambda b: (b, 0, 0)),
        out_shape=jax.ShapeDtypeStruct((B, MODEL_DIM, L), jnp.float32),
    )(W2, o_idx[:, :, None], vals[:, :, None], l_idx[:, :, None], b2c)


def kernel(x, W1, b1, W2, b2):
    b1c = b1[:, None]
    b2c = b2[:, None]

    h, gmax = _matmul1(x, W1, b1c)

    cand = _select_groups(gmax)
    slots = cand.reshape(B, TPB * CAP)
    top_slots, _ = lax.top_k(slots, NSEL)
    valid = top_slots >= 0
    gid = jnp.maximum(top_slots, 0)
    gm = gid // L
    gl = gid % L

    hf = h.reshape(B, HIGH_DIM * L)
    cf = ((gm[:, :, None] * GROUP + jnp.arange(GROUP)[None, None, :]) * L
          + gl[:, :, None])
    cf = jnp.where(valid[:, :, None], cf, HIGH_DIM * L).reshape(B, NSEL * GROUP)
    cf = jnp.sort(cf, axis=1)
    cv = jnp.take_along_axis(hf, jnp.minimum(cf, HIGH_DIM * L - 1), axis=1)
    cv = jnp.where(cf >= HIGH_DIM * L, -jnp.inf, cv)
    vals, pos = lax.top_k(cv, KEEP)
    idx = jnp.take_along_axis(cf, pos, axis=1)
    o_idx = idx // L
    l_idx = idx % L

    return _matmul2(W2, o_idx, vals, l_idx, b2c)
